# Initial kernel scaffold; baseline (speedup 1.0000x reference)
import jax
import jax.numpy as jnp
from jax.experimental import pallas as pl


def kernel(features, rbf_expansion, neighbor_list, neighbor_mask, W1, b1, W2, b2):
    raise NotImplementedError("write your pallas kernel here")


# speedup vs baseline: 650.0576x; 650.0576x over previous
import jax
import jax.numpy as jnp
from jax.experimental import pallas as pl

_TN = 40


def _fused_body(nl_ref, rbf_ref, mask_ref, feat_ref, w1_ref, b1_ref,
                w2_ref, b2_ref, out_ref):
    rows, g_dim = rbf_ref.shape[1], rbf_ref.shape[2]
    n = feat_ref.shape[1]
    f = feat_ref.shape[2]
    k = rows // out_ref.shape[1]

    rbf = rbf_ref[0]
    h = jnp.dot(rbf, w1_ref[...], preferred_element_type=jnp.float32)
    h = h + b1_ref[...]
    h = jax.nn.softplus(h) - jnp.log(2.0)
    filt = jnp.dot(h, w2_ref[...], preferred_element_type=jnp.float32)
    filt = (filt + b2_ref[...]) * mask_ref[0]

    nl = nl_ref[0]
    lane = jax.lax.broadcasted_iota(jnp.int32, (1, n), 1)
    onehot = (nl == lane).astype(jnp.bfloat16)
    gathered = jnp.dot(onehot, feat_ref[0],
                       preferred_element_type=jnp.float32)

    prod = filt * gathered
    out_ref[0] = prod.reshape(out_ref.shape[1], k, f).sum(axis=1)


def kernel(features, rbf_expansion, neighbor_list, neighbor_mask,
           W1, b1, W2, b2):
    B, N, F = features.shape
    _, _, K, G = rbf_expansion.shape
    tn = _TN
    rows = tn * K

    feat_bf = features.astype(jnp.bfloat16)
    nl = neighbor_list.reshape(B, N * K, 1)
    mask = neighbor_mask.reshape(B, N * K, 1)
    rbf = rbf_expansion.reshape(B, N * K, G)
    b1r = b1.reshape(1, F)
    b2r = b2.reshape(1, F)

    return pl.pallas_call(
        _fused_body,
        grid=(B, N // tn),
        in_specs=[
            pl.BlockSpec((1, rows, 1), lambda b, t: (b, t, 0)),
            pl.BlockSpec((1, rows, G), lambda b, t: (b, t, 0)),
            pl.BlockSpec((1, rows, 1), lambda b, t: (b, t, 0)),
            pl.BlockSpec((1, N, F), lambda b, t: (b, 0, 0)),
            pl.BlockSpec((G, F), lambda b, t: (0, 0)),
            pl.BlockSpec((1, F), lambda b, t: (0, 0)),
            pl.BlockSpec((F, F), lambda b, t: (0, 0)),
            pl.BlockSpec((1, F), lambda b, t: (0, 0)),
        ],
        out_specs=pl.BlockSpec((1, tn, F), lambda b, t: (b, t, 0)),
        out_shape=jax.ShapeDtypeStruct((B, N, F), jnp.float32),
    )(nl, rbf, mask, feat_bf, W1, b1r, W2, b2r)
